# speedup vs baseline: 2.1872x; 2.1872x over previous
import jax
import jax.numpy as jnp
from jax.experimental import pallas as pl


def _pos_block_kernel(row_ref, col_ref, dep_ref, out_ref):
    i = pl.program_id(1)
    W, F = col_ref.shape
    D = dep_ref.shape[0]
    row_vec = row_ref[pl.ds(i, 1), :]
    out_ref[0, :, 0:F] = jnp.broadcast_to(row_vec, (W * D, F))
    col_pat = jnp.broadcast_to(col_ref[:][:, None, :], (W, D, F)).reshape(W * D, F)
    out_ref[0, :, F:2 * F] = col_pat
    dep_pat = jnp.broadcast_to(dep_ref[:][None, :, :], (W, D, F)).reshape(W * D, F)
    out_ref[0, :, 2 * F:3 * F] = dep_pat


def kernel(B, h, w, d, x, row_embed, col_embed, dep_embed):
    H, F = row_embed.shape
    W = col_embed.shape[0]
    D = dep_embed.shape[0]
    Bs = x.shape[0]
    out = pl.pallas_call(
        _pos_block_kernel,
        grid=(Bs, H),
        in_specs=[
            pl.BlockSpec((H, F), lambda b, i: (0, 0)),
            pl.BlockSpec((W, F), lambda b, i: (0, 0)),
            pl.BlockSpec((D, F), lambda b, i: (0, 0)),
        ],
        out_specs=pl.BlockSpec((1, W * D, 3 * F), lambda b, i: (b, i, 0)),
        out_shape=jax.ShapeDtypeStruct((Bs, H * W * D, 3 * F), jnp.float32),
    )(row_embed, col_embed, dep_embed)
    return out
